# baseline (speedup 1.0000x reference)
import functools

import jax
import jax.numpy as jnp
from jax import lax
from jax.experimental import pallas as pl
from jax.experimental.pallas import tpu as pltpu
from jax.experimental.pallas import tpu_sc as plsc

N = 10000
E = 160000
D = 256
H = 128
NT = 16
EPT = E // NT
CH = 64
NCH = 160
NPH = 4
CPP = NCH // NPH
NB = 4
LA = 3
LG = CH // 16
EPT_PAD = NCH * CH
ACC_ROWS = NT * 640

_MESH = plsc.VectorSubcoreMesh(core_axis_name="c", subcore_axis_name="s")

_DIAG = None


def _issue_scatter(rows, acc, cacc, dstv, onesv, ssem, buf, j):
    if _DIAG == "gather-only":
        pltpu.async_copy(onesv, cacc.at[dstv.at[j]], ssem.at[buf], add=True)
    else:
        pltpu.async_copy(rows.at[buf], acc.at[dstv.at[j]], ssem.at[buf], add=True)


def _drain_scatter(rows, acc, cacc, dstv, onesv, ssem, buf):
    if _DIAG == "gather-only":
        pltpu.make_async_copy(onesv, cacc.at[dstv.at[0]], ssem.at[buf]).wait()
    else:
        pltpu.make_async_copy(rows.at[buf], acc.at[dstv.at[0]], ssem.at[buf]).wait()


def _seg_body(with_cnt, y_hbm, src_hbm, dst_hbm, out_hbm, cnt_hbm,
              acc, cacc, srcv, dstv, rows, onesv, cbuf, gsem, ssem, csem):
    c = lax.axis_index("c")
    t = lax.axis_index("s")

    zero16 = jnp.zeros((16,), jnp.float32)
    one16 = jnp.ones((16,), jnp.float32)

    nsub = H // 16
    def _zrows(i, _):
        rows[0, i // nsub, pl.ds(lax.rem(i, nsub) * 16, 16)] = zero16
        return 0
    lax.fori_loop(0, CH * nsub, _zrows, 0)
    for k in range(LG):
        onesv[pl.ds(k * 16, 16)] = one16
    for k in range(40):
        cbuf[pl.ds(k * 16, 16)] = zero16

    for k in range(640 // CH):
        pltpu.sync_copy(rows.at[0], acc.at[pl.ds(t * 640 + k * CH, CH)])
    if with_cnt:
        pltpu.sync_copy(cbuf, cacc.at[pl.ds(t * 640, 640)])
    plsc.subcore_barrier()

    cN16 = jnp.full((16,), c * N, jnp.int32)
    for phase in range(NPH):
        pltpu.sync_copy(src_hbm.at[t, pl.ds(phase * CPP, CPP)], srcv)
        pltpu.sync_copy(dst_hbm.at[t, pl.ds(phase * CPP, CPP)], dstv)

        def _shift(i, _):
            srcv[i // LG, pl.ds(lax.rem(i, LG) * 16, 16)] = (
                srcv[i // LG, pl.ds(lax.rem(i, LG) * 16, 16)] + cN16)
            return 0
        lax.fori_loop(0, CPP * LG, _shift, 0)

        def _grow(b):
            if _DIAG == "gather-half":
                return rows.at[b, :, pl.ds(0, 64)]
            return rows.at[b]
        if _DIAG != "scatter-only":
            for p in range(LA):
                pltpu.async_copy(y_hbm.at[srcv.at[p]], _grow(p), gsem.at[p])

        def _edge_chunk(j, _):
            buf = lax.rem(j, NB)
            abuf = lax.rem(j + LA, NB)
            @pl.when(j >= NB - LA)
            def _():
                _drain_scatter(rows, acc, cacc, dstv, onesv, ssem, abuf)
            if _DIAG != "scatter-only":
                @pl.when(j + LA < CPP)
                def _():
                    pltpu.async_copy(y_hbm.at[srcv.at[j + LA]], _grow(abuf),
                                     gsem.at[abuf])
                pltpu.make_async_copy(y_hbm.at[srcv.at[j]], _grow(buf),
                                      gsem.at[buf]).wait()
            _issue_scatter(rows, acc, cacc, dstv, onesv, ssem, buf, j)
            if with_cnt:
                @pl.when(c == 0)
                def _():
                    pltpu.async_copy(onesv, cacc.at[dstv.at[j]], csem, add=True)
            return 0
        lax.fori_loop(0, CPP, _edge_chunk, 0)
        for last in range(CPP - (NB - LA), CPP):
            _drain_scatter(rows, acc, cacc, dstv, onesv, ssem, last % NB)
        if with_cnt:
            @pl.when(c == 0)
            def _():
                def _drain(j, _):
                    pltpu.make_async_copy(onesv, cacc.at[dstv.at[0]], csem).wait()
                    return 0
                lax.fori_loop(0, CPP, _drain, 0)
    plsc.subcore_barrier()

    def _out_chunk(base, nrows):
        pltpu.sync_copy(acc.at[pl.ds(base, nrows)], rows.at[0, pl.ds(0, nrows)])
        pltpu.sync_copy(rows.at[0, pl.ds(0, nrows)], out_hbm.at[c, pl.ds(base, nrows)])

    @pl.when(t < NT - 1)
    def _():
        for k in range(640 // CH):
            _out_chunk(t * 640 + k * CH, CH)

    @pl.when(t == NT - 1)
    def _():
        for k in range(400 // CH):
            _out_chunk(t * 640 + k * CH, CH)
        _out_chunk(t * 640 + (400 // CH) * CH, 400 - (400 // CH) * CH)
    if with_cnt:
        @pl.when(c == 0)
        def _():
            @pl.when(t < NT - 1)
            def _():
                pltpu.sync_copy(cacc.at[pl.ds(t * 640, 640)], cbuf)
                pltpu.sync_copy(cbuf, cnt_hbm.at[pl.ds(t * 640, 640)])
            @pl.when(t == NT - 1)
            def _():
                pltpu.sync_copy(cacc.at[pl.ds(t * 640, 400)], cbuf.at[pl.ds(0, 400)])
                pltpu.sync_copy(cbuf.at[pl.ds(0, 400)], cnt_hbm.at[pl.ds(t * 640, 400)])


def _make_seg(with_cnt):
    outs = [jax.ShapeDtypeStruct((2, N, H), jnp.float32)]
    if with_cnt:
        outs.append(jax.ShapeDtypeStruct((N,), jnp.float32))
    scratch = [
        pltpu.VMEM_SHARED((ACC_ROWS, H), jnp.float32),
        pltpu.VMEM_SHARED((ACC_ROWS,), jnp.float32),
        pltpu.VMEM((CPP, CH), jnp.int32),
        pltpu.VMEM((CPP, CH), jnp.int32),
        pltpu.VMEM((NB, CH, H), jnp.float32),
        pltpu.VMEM((CH,), jnp.float32),
        pltpu.VMEM((640,), jnp.float32),
        pltpu.SemaphoreType.DMA((NB,)),
        pltpu.SemaphoreType.DMA((NB,)),
        pltpu.SemaphoreType.DMA,
    ]

    def body(y_hbm, src_hbm, dst_hbm, *rest):
        if with_cnt:
            out_hbm, cnt_hbm = rest[0], rest[1]
            rest = rest[2:]
        else:
            out_hbm, cnt_hbm = rest[0], None
            rest = rest[1:]
        _seg_body(with_cnt, y_hbm, src_hbm, dst_hbm, out_hbm, cnt_hbm, *rest)

    return pl.kernel(body, out_type=tuple(outs) if with_cnt else outs[0],
                     mesh=_MESH, scratch_types=scratch)


_seg_with_cnt = _make_seg(True)
_seg_no_cnt = _make_seg(False)


_RB = 1000


def _dense1_body(x_ref, wl_ref, wr_ref, b_ref, y_ref, xr_ref):
    xb = x_ref[...]
    y = jnp.dot(xb, wl_ref[...], preferred_element_type=jnp.float32)
    y_ref[0] = y[:, :H]
    y_ref[1] = y[:, H:]
    xr_ref[...] = jnp.dot(xb, wr_ref[...], preferred_element_type=jnp.float32) + b_ref[...]


def _mid_body(s_ref, cnt_ref, xr_ref, wl_ref, wr_ref, b_ref, y_ref, xr2_ref):
    cnt = jnp.maximum(cnt_ref[:, :1], 1.0)
    mean = jnp.concatenate([s_ref[0], s_ref[1]], axis=1) / cnt
    h = jnp.maximum(mean + xr_ref[...], 0.0)
    y = jnp.dot(h, wl_ref[...], preferred_element_type=jnp.float32)
    y_ref[0] = y[:, :H]
    y_ref[1] = y[:, H:]
    xr2_ref[...] = jnp.dot(h, wr_ref[...], preferred_element_type=jnp.float32) + b_ref[...]


def _final_body(s_ref, cnt_ref, xr_ref, o_ref):
    cnt = jnp.maximum(cnt_ref[:, :1], 1.0)
    o_ref[...] = jnp.concatenate([s_ref[0], s_ref[1]], axis=1) / cnt + xr_ref[...]


_spec_x = pl.BlockSpec((_RB, D), lambda i: (i, 0))
_spec_w = pl.BlockSpec((D, D), lambda i: (0, 0))
_spec_b = pl.BlockSpec((1, D), lambda i: (0, 0))
_spec_y = pl.BlockSpec((2, _RB, H), lambda i: (0, i, 0))
_spec_cnt = pl.BlockSpec((_RB, 8), lambda i: (i, 0))

_dense1 = pl.pallas_call(
    _dense1_body,
    grid=(N // _RB,),
    in_specs=[_spec_x, _spec_w, _spec_w, _spec_b],
    out_specs=[_spec_y, _spec_x],
    out_shape=[jax.ShapeDtypeStruct((2, N, H), jnp.float32),
               jax.ShapeDtypeStruct((N, D), jnp.float32)],
)

_mid = pl.pallas_call(
    _mid_body,
    grid=(N // _RB,),
    in_specs=[_spec_y, _spec_cnt, _spec_x, _spec_w, _spec_w, _spec_b],
    out_specs=[_spec_y, _spec_x],
    out_shape=[jax.ShapeDtypeStruct((2, N, H), jnp.float32),
               jax.ShapeDtypeStruct((N, D), jnp.float32)],
)

_final = pl.pallas_call(
    _final_body,
    grid=(N // _RB,),
    in_specs=[_spec_y, _spec_cnt, _spec_x],
    out_specs=_spec_x,
    out_shape=jax.ShapeDtypeStruct((N, D), jnp.float32),
)


def kernel(x, edge_index, W1l, b1, W1r, W2l, b2, W2r):
    src = edge_index[0]
    dst = edge_index[1]
    srcp = jnp.pad(src.reshape(NT, EPT), ((0, 0), (0, EPT_PAD - EPT))
                   ).reshape(NT, NCH, CH)
    dstp = jnp.pad(dst.reshape(NT, EPT), ((0, 0), (0, EPT_PAD - EPT)),
                   constant_values=N).reshape(NT, NCH, CH)

    y1, xr1 = _dense1(x, W1l.T, W1r.T, b1.reshape(1, D))
    s1, cnt = _seg_with_cnt(y1.reshape(2 * N, H), srcp, dstp)
    cnt8 = jnp.broadcast_to(cnt.reshape(N, 1), (N, 8))
    y2, xr2 = _mid(s1, cnt8, xr1, W2l.T, W2r.T, b2.reshape(1, D))
    s2 = _seg_no_cnt(y2.reshape(2 * N, H), srcp, dstp)
    return _final(s2, cnt8, xr2)


# speedup vs baseline: 1.0006x; 1.0006x over previous
import jax
import jax.numpy as jnp
from jax import lax
from jax.experimental import pallas as pl
from jax.experimental.pallas import tpu as pltpu
from jax.experimental.pallas import tpu_sc as plsc

N = 10000
E = 160000
D = 256
H = 128
NT = 16
EPT = E // NT
CH = 64
NCH = 160
NPH = 4
CPP = NCH // NPH
NB = 4
LA = 3
LG = CH // 16
EPT_PAD = NCH * CH
ACC_ROWS = NT * 640

_MESH = plsc.VectorSubcoreMesh(core_axis_name="c", subcore_axis_name="s")


def _seg_body(with_cnt, y_hbm, src_hbm, dst_hbm, out_hbm, cnt_hbm,
              acc, cacc, srcv, dstv, rows, onesv, cbuf, gsem, ssem, csem):
    c = lax.axis_index("c")
    t = lax.axis_index("s")

    zero16 = jnp.zeros((16,), jnp.float32)
    one16 = jnp.ones((16,), jnp.float32)

    nsub = H // 16
    def _zrows(i, _):
        rows[0, i // nsub, pl.ds(lax.rem(i, nsub) * 16, 16)] = zero16
        return 0
    lax.fori_loop(0, CH * nsub, _zrows, 0)
    for k in range(LG):
        onesv[pl.ds(k * 16, 16)] = one16
    for k in range(40):
        cbuf[pl.ds(k * 16, 16)] = zero16

    for k in range(640 // CH):
        pltpu.sync_copy(rows.at[0], acc.at[pl.ds(t * 640 + k * CH, CH)])
    if with_cnt:
        pltpu.sync_copy(cbuf, cacc.at[pl.ds(t * 640, 640)])
    plsc.subcore_barrier()

    cN16 = jnp.full((16,), c * N, jnp.int32)
    for phase in range(NPH):
        pltpu.sync_copy(src_hbm.at[t, pl.ds(phase * CPP, CPP)], srcv)
        pltpu.sync_copy(dst_hbm.at[t, pl.ds(phase * CPP, CPP)], dstv)

        def _shift(i, _):
            srcv[i // LG, pl.ds(lax.rem(i, LG) * 16, 16)] = (
                srcv[i // LG, pl.ds(lax.rem(i, LG) * 16, 16)] + cN16)
            return 0
        lax.fori_loop(0, CPP * LG, _shift, 0)

        for p in range(LA):
            pltpu.async_copy(y_hbm.at[srcv.at[p]], rows.at[p], gsem.at[p])

        def _edge_chunk(j, _):
            buf = lax.rem(j, NB)
            abuf = lax.rem(j + LA, NB)
            @pl.when(j >= NB - LA)
            def _():
                pltpu.make_async_copy(rows.at[abuf], acc.at[dstv.at[0]],
                                      ssem.at[abuf]).wait()
            @pl.when(j + LA < CPP)
            def _():
                pltpu.async_copy(y_hbm.at[srcv.at[j + LA]], rows.at[abuf],
                                 gsem.at[abuf])
            pltpu.make_async_copy(y_hbm.at[srcv.at[j]], rows.at[buf],
                                  gsem.at[buf]).wait()
            pltpu.async_copy(rows.at[buf], acc.at[dstv.at[j]], ssem.at[buf],
                             add=True)
            if with_cnt:
                @pl.when(c == 0)
                def _():
                    pltpu.async_copy(onesv, cacc.at[dstv.at[j]], csem, add=True)
            return 0
        lax.fori_loop(0, CPP, _edge_chunk, 0)
        for last in range(CPP - (NB - LA), CPP):
            pltpu.make_async_copy(rows.at[last % NB], acc.at[dstv.at[0]],
                                  ssem.at[last % NB]).wait()
        if with_cnt:
            @pl.when(c == 0)
            def _():
                def _drain(j, _):
                    pltpu.make_async_copy(onesv, cacc.at[dstv.at[0]], csem).wait()
                    return 0
                lax.fori_loop(0, CPP, _drain, 0)
    plsc.subcore_barrier()

    def _out_chunk(base, nrows):
        pltpu.sync_copy(acc.at[pl.ds(base, nrows)], rows.at[0, pl.ds(0, nrows)])
        pltpu.sync_copy(rows.at[0, pl.ds(0, nrows)], out_hbm.at[c, pl.ds(base, nrows)])

    @pl.when(t < NT - 1)
    def _():
        for k in range(640 // CH):
            _out_chunk(t * 640 + k * CH, CH)

    @pl.when(t == NT - 1)
    def _():
        for k in range(400 // CH):
            _out_chunk(t * 640 + k * CH, CH)
        _out_chunk(t * 640 + (400 // CH) * CH, 400 - (400 // CH) * CH)
    if with_cnt:
        @pl.when(c == 0)
        def _():
            @pl.when(t < NT - 1)
            def _():
                pltpu.sync_copy(cacc.at[pl.ds(t * 640, 640)], cbuf)
                pltpu.sync_copy(cbuf, cnt_hbm.at[pl.ds(t * 640, 640)])
            @pl.when(t == NT - 1)
            def _():
                pltpu.sync_copy(cacc.at[pl.ds(t * 640, 400)], cbuf.at[pl.ds(0, 400)])
                pltpu.sync_copy(cbuf.at[pl.ds(0, 400)], cnt_hbm.at[pl.ds(t * 640, 400)])


def _make_seg(with_cnt):
    outs = [jax.ShapeDtypeStruct((2, N, H), jnp.float32)]
    if with_cnt:
        outs.append(jax.ShapeDtypeStruct((N,), jnp.float32))
    scratch = [
        pltpu.VMEM_SHARED((ACC_ROWS, H), jnp.float32),
        pltpu.VMEM_SHARED((ACC_ROWS,), jnp.float32),
        pltpu.VMEM((CPP, CH), jnp.int32),
        pltpu.VMEM((CPP, CH), jnp.int32),
        pltpu.VMEM((NB, CH, H), jnp.float32),
        pltpu.VMEM((CH,), jnp.float32),
        pltpu.VMEM((640,), jnp.float32),
        pltpu.SemaphoreType.DMA((NB,)),
        pltpu.SemaphoreType.DMA((NB,)),
        pltpu.SemaphoreType.DMA,
    ]

    def body(y_hbm, src_hbm, dst_hbm, *rest):
        if with_cnt:
            out_hbm, cnt_hbm = rest[0], rest[1]
            rest = rest[2:]
        else:
            out_hbm, cnt_hbm = rest[0], None
            rest = rest[1:]
        _seg_body(with_cnt, y_hbm, src_hbm, dst_hbm, out_hbm, cnt_hbm, *rest)

    return pl.kernel(body, out_type=tuple(outs) if with_cnt else outs[0],
                     mesh=_MESH, scratch_types=scratch)


_seg_with_cnt = _make_seg(True)
_seg_no_cnt = _make_seg(False)


_RB = 1000


def _dense1_body(x_ref, wl_ref, wr_ref, b_ref, y_ref, xr_ref):
    xb = x_ref[...]
    y = jnp.dot(xb, wl_ref[...], preferred_element_type=jnp.float32)
    y_ref[0] = y[:, :H]
    y_ref[1] = y[:, H:]
    xr_ref[...] = jnp.dot(xb, wr_ref[...], preferred_element_type=jnp.float32) + b_ref[...]


def _mid_body(s_ref, cnt_ref, xr_ref, wl_ref, wr_ref, b_ref, y_ref, xr2_ref):
    cnt = jnp.maximum(cnt_ref[:, :1], 1.0)
    mean = jnp.concatenate([s_ref[0], s_ref[1]], axis=1) / cnt
    h = jnp.maximum(mean + xr_ref[...], 0.0)
    y = jnp.dot(h, wl_ref[...], preferred_element_type=jnp.float32)
    y_ref[0] = y[:, :H]
    y_ref[1] = y[:, H:]
    xr2_ref[...] = jnp.dot(h, wr_ref[...], preferred_element_type=jnp.float32) + b_ref[...]


def _final_body(s_ref, cnt_ref, xr_ref, o_ref):
    cnt = jnp.maximum(cnt_ref[:, :1], 1.0)
    o_ref[...] = jnp.concatenate([s_ref[0], s_ref[1]], axis=1) / cnt + xr_ref[...]


_spec_x = pl.BlockSpec((_RB, D), lambda i: (i, 0))
_spec_w = pl.BlockSpec((D, D), lambda i: (0, 0))
_spec_b = pl.BlockSpec((1, D), lambda i: (0, 0))
_spec_y = pl.BlockSpec((2, _RB, H), lambda i: (0, i, 0))
_spec_cnt = pl.BlockSpec((_RB, 8), lambda i: (i, 0))

_dense1 = pl.pallas_call(
    _dense1_body,
    grid=(N // _RB,),
    in_specs=[_spec_x, _spec_w, _spec_w, _spec_b],
    out_specs=[_spec_y, _spec_x],
    out_shape=[jax.ShapeDtypeStruct((2, N, H), jnp.float32),
               jax.ShapeDtypeStruct((N, D), jnp.float32)],
)

_mid = pl.pallas_call(
    _mid_body,
    grid=(N // _RB,),
    in_specs=[_spec_y, _spec_cnt, _spec_x, _spec_w, _spec_w, _spec_b],
    out_specs=[_spec_y, _spec_x],
    out_shape=[jax.ShapeDtypeStruct((2, N, H), jnp.float32),
               jax.ShapeDtypeStruct((N, D), jnp.float32)],
)

_final = pl.pallas_call(
    _final_body,
    grid=(N // _RB,),
    in_specs=[_spec_y, _spec_cnt, _spec_x],
    out_specs=_spec_x,
    out_shape=jax.ShapeDtypeStruct((N, D), jnp.float32),
)


def kernel(x, edge_index, W1l, b1, W1r, W2l, b2, W2r):
    src = edge_index[0]
    dst = edge_index[1]
    srcp = jnp.pad(src.reshape(NT, EPT), ((0, 0), (0, EPT_PAD - EPT))
                   ).reshape(NT, NCH, CH)
    dstp = jnp.pad(dst.reshape(NT, EPT), ((0, 0), (0, EPT_PAD - EPT)),
                   constant_values=N).reshape(NT, NCH, CH)

    y1, xr1 = _dense1(x, W1l.T, W1r.T, b1.reshape(1, D))
    s1, cnt = _seg_with_cnt(y1.reshape(2 * N, H), srcp, dstp)
    cnt8 = jnp.broadcast_to(cnt.reshape(N, 1), (N, 8))
    y2, xr2 = _mid(s1, cnt8, xr1, W2l.T, W2r.T, b2.reshape(1, D))
    s2 = _seg_no_cnt(y2.reshape(2 * N, H), srcp, dstp)
    return _final(s2, cnt8, xr2)


# speedup vs baseline: 1.0101x; 1.0095x over previous
import jax
import jax.numpy as jnp
from jax import lax
from jax.experimental import pallas as pl
from jax.experimental.pallas import tpu as pltpu
from jax.experimental.pallas import tpu_sc as plsc

N = 10000
E = 160000
D = 256
H = 128
NT = 16
EPT = E // NT
CH = 64
NCH = 160
NPH = 4
CPP = NCH // NPH
NB = 4
LA = 3
LG = CH // 16
EPT_PAD = NCH * CH
ACC_ROWS = NT * 640

_MESH = plsc.VectorSubcoreMesh(core_axis_name="c", subcore_axis_name="s")


def _seg_body(with_cnt, y_hbm, src_hbm, dst_hbm, out_hbm, cnt_hbm,
              acc, cacc, srcv, dstv, rows, onesv, cbuf, gsem, ssem, csem):
    c = lax.axis_index("c")
    t = lax.axis_index("s")

    zero16 = jnp.zeros((16,), jnp.float32)
    one16 = jnp.ones((16,), jnp.float32)

    nsub = H // 16
    def _zrows(i, _):
        rows[0, i // nsub, pl.ds(lax.rem(i, nsub) * 16, 16)] = zero16
        return 0
    lax.fori_loop(0, CH * nsub, _zrows, 0)
    for k in range(LG):
        onesv[pl.ds(k * 16, 16)] = one16
    for k in range(40):
        cbuf[pl.ds(k * 16, 16)] = zero16

    for k in range(640 // CH):
        pltpu.sync_copy(rows.at[0], acc.at[pl.ds(t * 640 + k * CH, CH)])
    if with_cnt:
        pltpu.sync_copy(cbuf, cacc.at[pl.ds(t * 640, 640)])
    plsc.subcore_barrier()

    cN16 = jnp.full((16,), c * N, jnp.int32)
    for phase in range(NPH):
        pltpu.sync_copy(src_hbm.at[t, pl.ds(phase * CPP, CPP)], srcv)
        pltpu.sync_copy(dst_hbm.at[t, pl.ds(phase * CPP, CPP)], dstv)

        def _shift(i, _):
            srcv[i // LG, pl.ds(lax.rem(i, LG) * 16, 16)] = (
                srcv[i // LG, pl.ds(lax.rem(i, LG) * 16, 16)] + cN16)
            return 0
        lax.fori_loop(0, CPP * LG, _shift, 0)

        for p in range(LA):
            pltpu.async_copy(y_hbm.at[srcv.at[p]], rows.at[p], gsem.at[p])

        def _edge_chunk(j, _):
            buf = lax.rem(j, NB)
            abuf = lax.rem(j + LA, NB)
            @pl.when(j >= NB - LA)
            def _():
                pltpu.make_async_copy(rows.at[abuf], acc.at[dstv.at[0]],
                                      ssem.at[abuf]).wait()
            @pl.when(j + LA < CPP)
            def _():
                pltpu.async_copy(y_hbm.at[srcv.at[j + LA]], rows.at[abuf],
                                 gsem.at[abuf])
            pltpu.make_async_copy(y_hbm.at[srcv.at[j]], rows.at[buf],
                                  gsem.at[buf]).wait()
            pltpu.async_copy(rows.at[buf], acc.at[dstv.at[j]], ssem.at[buf],
                             add=True)
            if with_cnt:
                @pl.when(c == 0)
                def _():
                    pltpu.async_copy(onesv, cacc.at[dstv.at[j]], csem, add=True)
            return 0
        lax.fori_loop(0, CPP, _edge_chunk, 0)
        for last in range(CPP - (NB - LA), CPP):
            pltpu.make_async_copy(rows.at[last % NB], acc.at[dstv.at[0]],
                                  ssem.at[last % NB]).wait()
        if with_cnt:
            @pl.when(c == 0)
            def _():
                def _drain(j, _):
                    pltpu.make_async_copy(onesv, cacc.at[dstv.at[0]], csem).wait()
                    return 0
                lax.fori_loop(0, CPP, _drain, 0)
    plsc.subcore_barrier()

    def _out_chunk(base, nrows):
        pltpu.sync_copy(acc.at[pl.ds(base, nrows)], rows.at[0, pl.ds(0, nrows)])
        pltpu.sync_copy(rows.at[0, pl.ds(0, nrows)], out_hbm.at[c, pl.ds(base, nrows)])

    @pl.when(t < NT - 1)
    def _():
        for k in range(640 // CH):
            _out_chunk(t * 640 + k * CH, CH)

    @pl.when(t == NT - 1)
    def _():
        for k in range(400 // CH):
            _out_chunk(t * 640 + k * CH, CH)
        _out_chunk(t * 640 + (400 // CH) * CH, 400 - (400 // CH) * CH)
    if with_cnt:
        @pl.when(c == 0)
        def _():
            @pl.when(t < NT - 1)
            def _():
                pltpu.sync_copy(cacc.at[pl.ds(t * 640, 640)], cbuf)
                pltpu.sync_copy(cbuf, cnt_hbm.at[pl.ds(t * 640, 640)])
            @pl.when(t == NT - 1)
            def _():
                pltpu.sync_copy(cacc.at[pl.ds(t * 640, 400)], cbuf.at[pl.ds(0, 400)])
                pltpu.sync_copy(cbuf.at[pl.ds(0, 400)], cnt_hbm.at[pl.ds(t * 640, 400)])


def _make_seg(with_cnt):
    outs = [jax.ShapeDtypeStruct((2, N, H), jnp.float32)]
    if with_cnt:
        outs.append(jax.ShapeDtypeStruct((N,), jnp.float32))
    scratch = [
        pltpu.VMEM_SHARED((ACC_ROWS, H), jnp.float32),
        pltpu.VMEM_SHARED((ACC_ROWS,), jnp.float32),
        pltpu.VMEM((CPP, CH), jnp.int32),
        pltpu.VMEM((CPP, CH), jnp.int32),
        pltpu.VMEM((NB, CH, H), jnp.float32),
        pltpu.VMEM((CH,), jnp.float32),
        pltpu.VMEM((640,), jnp.float32),
        pltpu.SemaphoreType.DMA((NB,)),
        pltpu.SemaphoreType.DMA((NB,)),
        pltpu.SemaphoreType.DMA,
    ]

    def body(y_hbm, src_hbm, dst_hbm, *rest):
        if with_cnt:
            out_hbm, cnt_hbm = rest[0], rest[1]
            rest = rest[2:]
        else:
            out_hbm, cnt_hbm = rest[0], None
            rest = rest[1:]
        _seg_body(with_cnt, y_hbm, src_hbm, dst_hbm, out_hbm, cnt_hbm, *rest)

    return pl.kernel(body, out_type=tuple(outs) if with_cnt else outs[0],
                     mesh=_MESH, scratch_types=scratch)


_seg_with_cnt = _make_seg(True)
_seg_no_cnt = _make_seg(False)


_RB = 1000


def _proj_body(x_ref, wl_ref, y_ref):
    y = jnp.dot(x_ref[...], wl_ref[...], preferred_element_type=jnp.float32)
    y_ref[0] = y[:, :H]
    y_ref[1] = y[:, H:]


def _projb_body(x_ref, wr_ref, b_ref, xr_ref):
    xr_ref[...] = jnp.dot(x_ref[...], wr_ref[...],
                          preferred_element_type=jnp.float32) + b_ref[...]


def _mid_y_body(s_ref, cnt_ref, xr_ref, wl_ref, y_ref):
    cnt = jnp.maximum(cnt_ref[:, :1], 1.0)
    h = jnp.maximum(jnp.concatenate([s_ref[0], s_ref[1]], axis=1) / cnt
                    + xr_ref[...], 0.0)
    y = jnp.dot(h, wl_ref[...], preferred_element_type=jnp.float32)
    y_ref[0] = y[:, :H]
    y_ref[1] = y[:, H:]


def _mid_xr_body(s_ref, cnt_ref, xr_ref, wr_ref, b_ref, xr2_ref):
    cnt = jnp.maximum(cnt_ref[:, :1], 1.0)
    h = jnp.maximum(jnp.concatenate([s_ref[0], s_ref[1]], axis=1) / cnt
                    + xr_ref[...], 0.0)
    xr2_ref[...] = jnp.dot(h, wr_ref[...],
                           preferred_element_type=jnp.float32) + b_ref[...]


def _final_body(s_ref, cnt_ref, xr_ref, o_ref):
    cnt = jnp.maximum(cnt_ref[:, :1], 1.0)
    o_ref[...] = jnp.concatenate([s_ref[0], s_ref[1]], axis=1) / cnt + xr_ref[...]


_spec_x = pl.BlockSpec((_RB, D), lambda i: (i, 0))
_spec_w = pl.BlockSpec((D, D), lambda i: (0, 0))
_spec_b = pl.BlockSpec((1, D), lambda i: (0, 0))
_spec_y = pl.BlockSpec((2, _RB, H), lambda i: (0, i, 0))
_spec_cnt = pl.BlockSpec((_RB, 8), lambda i: (i, 0))

_proj = pl.pallas_call(
    _proj_body,
    grid=(N // _RB,),
    in_specs=[_spec_x, _spec_w],
    out_specs=_spec_y,
    out_shape=jax.ShapeDtypeStruct((2, N, H), jnp.float32),
)

_projb = pl.pallas_call(
    _projb_body,
    grid=(N // _RB,),
    in_specs=[_spec_x, _spec_w, _spec_b],
    out_specs=_spec_x,
    out_shape=jax.ShapeDtypeStruct((N, D), jnp.float32),
)

_mid_y = pl.pallas_call(
    _mid_y_body,
    grid=(N // _RB,),
    in_specs=[_spec_y, _spec_cnt, _spec_x, _spec_w],
    out_specs=_spec_y,
    out_shape=jax.ShapeDtypeStruct((2, N, H), jnp.float32),
)

_mid_xr = pl.pallas_call(
    _mid_xr_body,
    grid=(N // _RB,),
    in_specs=[_spec_y, _spec_cnt, _spec_x, _spec_w, _spec_b],
    out_specs=_spec_x,
    out_shape=jax.ShapeDtypeStruct((N, D), jnp.float32),
)

_final = pl.pallas_call(
    _final_body,
    grid=(N // _RB,),
    in_specs=[_spec_y, _spec_cnt, _spec_x],
    out_specs=_spec_x,
    out_shape=jax.ShapeDtypeStruct((N, D), jnp.float32),
)


def kernel(x, edge_index, W1l, b1, W1r, W2l, b2, W2r):
    src = edge_index[0]
    dst = edge_index[1]
    srcp = jnp.pad(src.reshape(NT, EPT), ((0, 0), (0, EPT_PAD - EPT))
                   ).reshape(NT, NCH, CH)
    dstp = jnp.pad(dst.reshape(NT, EPT), ((0, 0), (0, EPT_PAD - EPT)),
                   constant_values=N).reshape(NT, NCH, CH)

    y1 = _proj(x, W1l.T)
    s1, cnt = _seg_with_cnt(y1.reshape(2 * N, H), srcp, dstp)
    xr1 = _projb(x, W1r.T, b1.reshape(1, D))
    cnt8 = jnp.broadcast_to(cnt.reshape(N, 1), (N, 8))
    y2 = _mid_y(s1, cnt8, xr1, W2l.T)
    s2 = _seg_no_cnt(y2.reshape(2 * N, H), srcp, dstp)
    xr2 = _mid_xr(s1, cnt8, xr1, W2r.T, b2.reshape(1, D))
    return _final(s2, cnt8, xr2)
